# baseline (speedup 1.0000x reference)
import functools

import jax
import jax.numpy as jnp
from jax import lax
from jax.experimental import pallas as pl
from jax.experimental.pallas import tpu as pltpu
from jax.experimental.pallas import tpu_sc as plsc

N = 10000
E = 320000
D = 128
NGRAPH = 128
NCLASS = 16

NC = 2
NS = 16
NW = NC * NS
CHUNK = 128
K = 80
KH = K // 2
E_PAD = NW * K * CHUNK
NBUF = 2
ACC_ROWS = N + 64
RPT = 624
LAST_ROWS = N - 15 * RPT
LAST_ZROWS = ACC_ROWS - 15 * RPT

ROW_BLK = 1000
GRID = N // ROW_BLK


def _sc_agg_body(h_hbm, z_hbm, src_hbm, dst_hbm, out_hbm,
                 acc, src_idx, dst_idx, rows, g0):
    cid = lax.axis_index("c")
    sid = lax.axis_index("s")
    wid = sid * NC + cid

    @pl.when(cid == 0)
    def _():
        @pl.when(sid < 15)
        def _():
            pltpu.sync_copy(h_hbm.at[pl.ds(sid * RPT, RPT)],
                            acc.at[pl.ds(sid * RPT, RPT)])

        @pl.when(sid == 15)
        def _():
            pltpu.sync_copy(h_hbm.at[pl.ds(15 * RPT, LAST_ROWS)],
                            acc.at[pl.ds(15 * RPT, LAST_ROWS)])
            pltpu.sync_copy(z_hbm.at[pl.ds(0, ACC_ROWS - N)],
                            acc.at[pl.ds(N, ACC_ROWS - N)])

    @pl.when(cid == 1)
    def _():
        @pl.when(sid < 15)
        def _():
            pltpu.sync_copy(z_hbm.at[pl.ds(sid * RPT, RPT)],
                            acc.at[pl.ds(sid * RPT, RPT)])

        @pl.when(sid == 15)
        def _():
            pltpu.sync_copy(z_hbm.at[pl.ds(15 * RPT, LAST_ZROWS)],
                            acc.at[pl.ds(15 * RPT, LAST_ZROWS)])

    plsc.subcore_barrier()

    def step(j):
        base = (wid * K + j) * CHUNK
        pltpu.sync_copy(src_hbm.at[pl.ds(base, CHUNK)], src_idx)
        pltpu.sync_copy(dst_hbm.at[pl.ds(base, CHUNK)], dst_idx)
        pltpu.async_copy(h_hbm.at[src_idx], rows, g0).wait()
        pltpu.sync_copy(rows, acc.at[dst_idx], add=True)

    pl.loop(0, K, unroll=1)(step)

    plsc.subcore_barrier()

    @pl.when(sid < 15)
    def _():
        pltpu.sync_copy(acc.at[pl.ds(sid * RPT, RPT)],
                        out_hbm.at[cid, pl.ds(sid * RPT, RPT)])

    @pl.when(sid == 15)
    def _():
        pltpu.sync_copy(acc.at[pl.ds(15 * RPT, LAST_ROWS)],
                        out_hbm.at[cid, pl.ds(15 * RPT, LAST_ROWS)])


_sc_agg = pl.kernel(
    _sc_agg_body,
    out_type=jax.ShapeDtypeStruct((NC, N, D), jnp.float32),
    mesh=plsc.VectorSubcoreMesh(core_axis_name="c", subcore_axis_name="s"),
    scratch_types=[
        pltpu.VMEM_SHARED((ACC_ROWS, D), jnp.float32),
        pltpu.VMEM((CHUNK,), jnp.int32),
        pltpu.VMEM((CHUNK,), jnp.int32),
        pltpu.VMEM((CHUNK, D), jnp.float32),
        pltpu.SemaphoreType.DMA,
    ],
)


def _pre_body(x_ref, w_ref, b_ref, o_ref):
    o_ref[...] = jnp.dot(x_ref[...], w_ref[...],
                         preferred_element_type=jnp.float32) + b_ref[...]


def _pre_matmul(x, w, b):
    return pl.pallas_call(
        _pre_body,
        grid=(GRID,),
        in_specs=[
            pl.BlockSpec((ROW_BLK, D), lambda i: (i, 0)),
            pl.BlockSpec((D, D), lambda i: (0, 0)),
            pl.BlockSpec((1, D), lambda i: (0, 0)),
        ],
        out_specs=pl.BlockSpec((ROW_BLK, D), lambda i: (i, 0)),
        out_shape=jax.ShapeDtypeStruct((N, D), jnp.float32),
    )(x, w, b.reshape(1, D))


def _mlp_body(p_ref, wa_ref, ba_ref, wb_ref, bb_ref, o_ref):
    t = p_ref[0] + p_ref[1]
    u = jnp.maximum(jnp.dot(t, wa_ref[...],
                            preferred_element_type=jnp.float32) + ba_ref[...],
                    0.0)
    v = jnp.dot(u, wb_ref[...], preferred_element_type=jnp.float32) + bb_ref[...]
    o_ref[...] = jnp.maximum(v, 0.0)


def _mlp(parts, wa, ba, wb, bb):
    return pl.pallas_call(
        _mlp_body,
        grid=(GRID,),
        in_specs=[
            pl.BlockSpec((NC, ROW_BLK, D), lambda i: (0, i, 0)),
            pl.BlockSpec((D, D), lambda i: (0, 0)),
            pl.BlockSpec((1, D), lambda i: (0, 0)),
            pl.BlockSpec((D, D), lambda i: (0, 0)),
            pl.BlockSpec((1, D), lambda i: (0, 0)),
        ],
        out_specs=pl.BlockSpec((ROW_BLK, D), lambda i: (i, 0)),
        out_shape=jax.ShapeDtypeStruct((N, D), jnp.float32),
    )(parts, wa, ba.reshape(1, D), wb, bb.reshape(1, D))


def _pool_head_body(h_ref, seg_ref, wp_ref, bp_ref, wr_ref, br_ref,
                    o_ref, pool_ref):
    i = pl.program_id(0)

    @pl.when(i == 0)
    def _():
        pool_ref[...] = jnp.zeros_like(pool_ref)

    seg = seg_ref[0, 0, :]
    gids = lax.broadcasted_iota(jnp.int32, (NGRAPH, ROW_BLK), 0)
    onehot = (gids == seg[None, :]).astype(jnp.float32)
    pool_ref[...] += jnp.dot(onehot, h_ref[...],
                             preferred_element_type=jnp.float32)

    @pl.when(i == GRID - 1)
    def _():
        hp = jnp.maximum(jnp.dot(pool_ref[...], wp_ref[...],
                                 preferred_element_type=jnp.float32)
                         + bp_ref[...], 0.0)
        logits = jnp.dot(hp, wr_ref[...],
                         preferred_element_type=jnp.float32) + br_ref[...]
        m = jnp.max(logits, axis=1, keepdims=True)
        lse = m + jnp.log(jnp.sum(jnp.exp(logits - m), axis=1, keepdims=True))
        o_ref[...] = logits - lse


def _pool_head(h, seg3, wp, bp, wr, br):
    return pl.pallas_call(
        _pool_head_body,
        grid=(GRID,),
        in_specs=[
            pl.BlockSpec((ROW_BLK, D), lambda i: (i, 0)),
            pl.BlockSpec((1, 1, ROW_BLK), lambda i: (i, 0, 0)),
            pl.BlockSpec((D, D), lambda i: (0, 0)),
            pl.BlockSpec((1, D), lambda i: (0, 0)),
            pl.BlockSpec((D, NCLASS), lambda i: (0, 0)),
            pl.BlockSpec((1, NCLASS), lambda i: (0, 0)),
        ],
        out_specs=pl.BlockSpec((NGRAPH, NCLASS), lambda i: (0, 0)),
        out_shape=jax.ShapeDtypeStruct((NGRAPH, NCLASS), jnp.float32),
        scratch_shapes=[pltpu.VMEM((NGRAPH, D), jnp.float32)],
    )(h, seg3, wp, bp.reshape(1, D), wr, br.reshape(1, NCLASS))


def kernel(x, edge_index, batch, W_pre, b_pre, W0a, b0a, W0b, b0b,
           W1a, b1a, W1b, b1b, W2a, b2a, W2b, b2b,
           W_post, b_post, W_read, b_read):
    src = edge_index[0].astype(jnp.int32)
    dst = edge_index[1].astype(jnp.int32)
    pad = E_PAD - E
    src_p = jnp.concatenate([src, jnp.zeros((pad,), jnp.int32)])
    dst_p = jnp.concatenate(
        [dst, N + (jnp.arange(pad, dtype=jnp.int32) % 64)])
    zeros = jnp.zeros((ACC_ROWS, D), jnp.float32)
    seg3 = batch.astype(jnp.int32).reshape(GRID, 1, ROW_BLK)

    h = _pre_matmul(x, W_pre, b_pre)
    for (wa, ba, wb, bb) in ((W0a, b0a, W0b, b0b),
                             (W1a, b1a, W1b, b1b),
                             (W2a, b2a, W2b, b2b)):
        parts = _sc_agg(h, zeros, src_p, dst_p)
        h = _mlp(parts, wa, ba, wb, bb)
    return _pool_head(h, seg3, W_post, b_post, W_read, b_read)


# speedup vs baseline: 1.4934x; 1.4934x over previous
import functools

import jax
import jax.numpy as jnp
from jax import lax
from jax.experimental import pallas as pl
from jax.experimental.pallas import tpu as pltpu
from jax.experimental.pallas import tpu_sc as plsc

N = 10000
E = 320000
D = 128
NGRAPH = 128
NCLASS = 16

NC = 2
NS = 16
NW = NC * NS
CHUNK = 128
K = 79
KH = K // 2
E_PAD = NW * K * CHUNK
NBUF = 2
ACC_ROWS = N + 64
RPT = 624
LAST_ROWS = N - 15 * RPT
LAST_ZROWS = ACC_ROWS - 15 * RPT

ROW_BLK = 1000
GRID = N // ROW_BLK


def _sc_agg_body(h_hbm, z_hbm, src_hbm, dst_hbm, out_hbm,
                 acc, src_idx, dst_idx, rows, g0):
    cid = lax.axis_index("c")
    sid = lax.axis_index("s")
    wid = sid * NC + cid

    @pl.when(cid == 0)
    def _():
        @pl.when(sid < 15)
        def _():
            pltpu.sync_copy(h_hbm.at[pl.ds(sid * RPT, RPT)],
                            acc.at[pl.ds(sid * RPT, RPT)])

        @pl.when(sid == 15)
        def _():
            pltpu.sync_copy(h_hbm.at[pl.ds(15 * RPT, LAST_ROWS)],
                            acc.at[pl.ds(15 * RPT, LAST_ROWS)])
            pltpu.sync_copy(z_hbm.at[pl.ds(0, ACC_ROWS - N)],
                            acc.at[pl.ds(N, ACC_ROWS - N)])

    @pl.when(cid == 1)
    def _():
        @pl.when(sid < 15)
        def _():
            pltpu.sync_copy(z_hbm.at[pl.ds(sid * RPT, RPT)],
                            acc.at[pl.ds(sid * RPT, RPT)])

        @pl.when(sid == 15)
        def _():
            pltpu.sync_copy(z_hbm.at[pl.ds(15 * RPT, LAST_ZROWS)],
                            acc.at[pl.ds(15 * RPT, LAST_ZROWS)])

    plsc.subcore_barrier()

    def step(j):
        base = (wid * K + j) * CHUNK
        pltpu.sync_copy(src_hbm.at[pl.ds(base, CHUNK)], src_idx)
        pltpu.sync_copy(dst_hbm.at[pl.ds(base, CHUNK)], dst_idx)
        pltpu.async_copy(h_hbm.at[src_idx], rows, g0).wait()
        pltpu.sync_copy(rows, acc.at[dst_idx], add=True)

    pl.loop(0, K)(step)

    plsc.subcore_barrier()

    @pl.when(sid < 15)
    def _():
        pltpu.sync_copy(acc.at[pl.ds(sid * RPT, RPT)],
                        out_hbm.at[cid, pl.ds(sid * RPT, RPT)])

    @pl.when(sid == 15)
    def _():
        pltpu.sync_copy(acc.at[pl.ds(15 * RPT, LAST_ROWS)],
                        out_hbm.at[cid, pl.ds(15 * RPT, LAST_ROWS)])


_sc_agg = pl.kernel(
    _sc_agg_body,
    out_type=jax.ShapeDtypeStruct((NC, N, D), jnp.float32),
    mesh=plsc.VectorSubcoreMesh(core_axis_name="c", subcore_axis_name="s"),
    scratch_types=[
        pltpu.VMEM_SHARED((ACC_ROWS, D), jnp.float32),
        pltpu.VMEM((CHUNK,), jnp.int32),
        pltpu.VMEM((CHUNK,), jnp.int32),
        pltpu.VMEM((CHUNK, D), jnp.float32),
        pltpu.SemaphoreType.DMA,
    ],
)


def _pre_body(x_ref, w_ref, b_ref, o_ref):
    o_ref[...] = jnp.dot(x_ref[...], w_ref[...],
                         preferred_element_type=jnp.float32) + b_ref[...]


def _pre_matmul(x, w, b):
    return pl.pallas_call(
        _pre_body,
        grid=(GRID,),
        in_specs=[
            pl.BlockSpec((ROW_BLK, D), lambda i: (i, 0)),
            pl.BlockSpec((D, D), lambda i: (0, 0)),
            pl.BlockSpec((1, D), lambda i: (0, 0)),
        ],
        out_specs=pl.BlockSpec((ROW_BLK, D), lambda i: (i, 0)),
        out_shape=jax.ShapeDtypeStruct((N, D), jnp.float32),
    )(x, w, b.reshape(1, D))


def _mlp_body(p_ref, wa_ref, ba_ref, wb_ref, bb_ref, o_ref):
    t = p_ref[0] + p_ref[1]
    u = jnp.maximum(jnp.dot(t, wa_ref[...],
                            preferred_element_type=jnp.float32) + ba_ref[...],
                    0.0)
    v = jnp.dot(u, wb_ref[...], preferred_element_type=jnp.float32) + bb_ref[...]
    o_ref[...] = jnp.maximum(v, 0.0)


def _mlp(parts, wa, ba, wb, bb):
    return pl.pallas_call(
        _mlp_body,
        grid=(GRID,),
        in_specs=[
            pl.BlockSpec((NC, ROW_BLK, D), lambda i: (0, i, 0)),
            pl.BlockSpec((D, D), lambda i: (0, 0)),
            pl.BlockSpec((1, D), lambda i: (0, 0)),
            pl.BlockSpec((D, D), lambda i: (0, 0)),
            pl.BlockSpec((1, D), lambda i: (0, 0)),
        ],
        out_specs=pl.BlockSpec((ROW_BLK, D), lambda i: (i, 0)),
        out_shape=jax.ShapeDtypeStruct((N, D), jnp.float32),
    )(parts, wa, ba.reshape(1, D), wb, bb.reshape(1, D))


def _pool_head_body(h_ref, seg_ref, wp_ref, bp_ref, wr_ref, br_ref,
                    o_ref, pool_ref):
    i = pl.program_id(0)

    @pl.when(i == 0)
    def _():
        pool_ref[...] = jnp.zeros_like(pool_ref)

    seg = seg_ref[0, 0, :]
    gids = lax.broadcasted_iota(jnp.int32, (NGRAPH, ROW_BLK), 0)
    onehot = (gids == seg[None, :]).astype(jnp.float32)
    pool_ref[...] += jnp.dot(onehot, h_ref[...],
                             preferred_element_type=jnp.float32)

    @pl.when(i == GRID - 1)
    def _():
        hp = jnp.maximum(jnp.dot(pool_ref[...], wp_ref[...],
                                 preferred_element_type=jnp.float32)
                         + bp_ref[...], 0.0)
        logits = jnp.dot(hp, wr_ref[...],
                         preferred_element_type=jnp.float32) + br_ref[...]
        m = jnp.max(logits, axis=1, keepdims=True)
        lse = m + jnp.log(jnp.sum(jnp.exp(logits - m), axis=1, keepdims=True))
        o_ref[...] = logits - lse


def _pool_head(h, seg3, wp, bp, wr, br):
    return pl.pallas_call(
        _pool_head_body,
        grid=(GRID,),
        in_specs=[
            pl.BlockSpec((ROW_BLK, D), lambda i: (i, 0)),
            pl.BlockSpec((1, 1, ROW_BLK), lambda i: (i, 0, 0)),
            pl.BlockSpec((D, D), lambda i: (0, 0)),
            pl.BlockSpec((1, D), lambda i: (0, 0)),
            pl.BlockSpec((D, NCLASS), lambda i: (0, 0)),
            pl.BlockSpec((1, NCLASS), lambda i: (0, 0)),
        ],
        out_specs=pl.BlockSpec((NGRAPH, NCLASS), lambda i: (0, 0)),
        out_shape=jax.ShapeDtypeStruct((NGRAPH, NCLASS), jnp.float32),
        scratch_shapes=[pltpu.VMEM((NGRAPH, D), jnp.float32)],
    )(h, seg3, wp, bp.reshape(1, D), wr, br.reshape(1, NCLASS))


def kernel(x, edge_index, batch, W_pre, b_pre, W0a, b0a, W0b, b0b,
           W1a, b1a, W1b, b1b, W2a, b2a, W2b, b2b,
           W_post, b_post, W_read, b_read):
    src = edge_index[0].astype(jnp.int32)
    dst = edge_index[1].astype(jnp.int32)
    pad = E_PAD - E
    src_p = jnp.concatenate([src, jnp.zeros((pad,), jnp.int32)])
    dst_p = jnp.concatenate(
        [dst, N + (jnp.arange(pad, dtype=jnp.int32) % 64)])
    zeros = jnp.zeros((ACC_ROWS, D), jnp.float32)
    seg3 = batch.astype(jnp.int32).reshape(GRID, 1, ROW_BLK)

    h = _pre_matmul(x, W_pre, b_pre)
    for (wa, ba, wb, bb) in ((W0a, b0a, W0b, b0b),
                             (W1a, b1a, W1b, b1b),
                             (W2a, b2a, W2b, b2b)):
        parts = _sc_agg(h, zeros, src_p, dst_p)
        h = _mlp(parts, wa, ba, wb, bb)
    return _pool_head(h, seg3, W_post, b_post, W_read, b_read)


# speedup vs baseline: 2.3269x; 1.5582x over previous
import functools

import jax
import jax.numpy as jnp
from jax import lax
from jax.experimental import pallas as pl
from jax.experimental.pallas import tpu as pltpu
from jax.experimental.pallas import tpu_sc as plsc

N = 10000
E = 320000
D = 128
NGRAPH = 128
NCLASS = 16

NC = 2
NS = 16
NW = NC * NS
CHUNK = 128
K = 79
KH = K // 2
E_PAD = NW * K * CHUNK
NBUF = 2
ACC_ROWS = N + 64
RPT = 624
LAST_ROWS = N - 15 * RPT
LAST_ZROWS = ACC_ROWS - 15 * RPT

ROW_BLK = 1000
GRID = N // ROW_BLK


def _sc_agg_body(h_hbm, z_hbm, src_hbm, dst_hbm, out_hbm,
                 acc, src_idx, dst_idx, rows, g0):
    cid = lax.axis_index("c")
    sid = lax.axis_index("s")
    wid = sid * NC + cid

    @pl.when(cid == 0)
    def _():
        @pl.when(sid < 15)
        def _():
            pltpu.sync_copy(h_hbm.at[pl.ds(sid * RPT, RPT)],
                            acc.at[pl.ds(sid * RPT, RPT)])

        @pl.when(sid == 15)
        def _():
            pltpu.sync_copy(h_hbm.at[pl.ds(15 * RPT, LAST_ROWS)],
                            acc.at[pl.ds(15 * RPT, LAST_ROWS)])
            pltpu.sync_copy(z_hbm.at[pl.ds(0, ACC_ROWS - N)],
                            acc.at[pl.ds(N, ACC_ROWS - N)])

    @pl.when(cid == 1)
    def _():
        @pl.when(sid < 15)
        def _():
            pltpu.sync_copy(z_hbm.at[pl.ds(sid * RPT, RPT)],
                            acc.at[pl.ds(sid * RPT, RPT)])

        @pl.when(sid == 15)
        def _():
            pltpu.sync_copy(z_hbm.at[pl.ds(15 * RPT, LAST_ZROWS)],
                            acc.at[pl.ds(15 * RPT, LAST_ZROWS)])

    plsc.subcore_barrier()

    def step(j):
        base = (wid * K + j) * CHUNK
        pltpu.sync_copy(src_hbm.at[pl.ds(base, CHUNK)], src_idx)
        pltpu.sync_copy(dst_hbm.at[pl.ds(base, CHUNK)], dst_idx)
        pltpu.async_copy(h_hbm.at[src_idx], rows, g0).wait()
        pltpu.sync_copy(rows, acc.at[dst_idx], add=True)

    pl.loop(0, K)(step)

    plsc.subcore_barrier()

    @pl.when(sid < 15)
    def _():
        pltpu.sync_copy(acc.at[pl.ds(sid * RPT, RPT)],
                        out_hbm.at[cid, pl.ds(sid * RPT, RPT)])

    @pl.when(sid == 15)
    def _():
        pltpu.sync_copy(acc.at[pl.ds(15 * RPT, LAST_ROWS)],
                        out_hbm.at[cid, pl.ds(15 * RPT, LAST_ROWS)])


_sc_agg = pl.kernel(
    _sc_agg_body,
    out_type=jax.ShapeDtypeStruct((NC, N, D), jnp.float32),
    mesh=plsc.VectorSubcoreMesh(core_axis_name="c", subcore_axis_name="s"),
    scratch_types=[
        pltpu.VMEM_SHARED((ACC_ROWS, D), jnp.float32),
        pltpu.VMEM((CHUNK,), jnp.int32),
        pltpu.VMEM((CHUNK,), jnp.int32),
        pltpu.VMEM((CHUNK, D), jnp.float32),
        pltpu.SemaphoreType.DMA,
    ],
)


def _pre_body(x_ref, w_ref, b_ref, o_ref):
    o_ref[...] = jnp.dot(x_ref[...], w_ref[...],
                         preferred_element_type=jnp.float32) + b_ref[...]


def _pre_matmul(x, w, b):
    return pl.pallas_call(
        _pre_body,
        grid=(GRID,),
        in_specs=[
            pl.BlockSpec((ROW_BLK, D), lambda i: (i, 0)),
            pl.BlockSpec((D, D), lambda i: (0, 0)),
            pl.BlockSpec((1, D), lambda i: (0, 0)),
        ],
        out_specs=pl.BlockSpec((ROW_BLK, D), lambda i: (i, 0)),
        out_shape=jax.ShapeDtypeStruct((N, D), jnp.float32),
    )(x, w, b.reshape(1, D))


def _mlp_body(p_ref, wa_ref, ba_ref, wb_ref, bb_ref, o_ref):
    t = p_ref[0] + p_ref[1]
    u = jnp.maximum(jnp.dot(t, wa_ref[...],
                            preferred_element_type=jnp.float32) + ba_ref[...],
                    0.0)
    v = jnp.dot(u, wb_ref[...], preferred_element_type=jnp.float32) + bb_ref[...]
    o_ref[...] = jnp.maximum(v, 0.0)


def _mlp(parts, wa, ba, wb, bb):
    return pl.pallas_call(
        _mlp_body,
        grid=(GRID,),
        in_specs=[
            pl.BlockSpec((NC, ROW_BLK, D), lambda i: (0, i, 0)),
            pl.BlockSpec((D, D), lambda i: (0, 0)),
            pl.BlockSpec((1, D), lambda i: (0, 0)),
            pl.BlockSpec((D, D), lambda i: (0, 0)),
            pl.BlockSpec((1, D), lambda i: (0, 0)),
        ],
        out_specs=pl.BlockSpec((ROW_BLK, D), lambda i: (i, 0)),
        out_shape=jax.ShapeDtypeStruct((N, D), jnp.float32),
    )(parts, wa, ba.reshape(1, D), wb, bb.reshape(1, D))


def _pool_head_body(h_ref, seg_ref, wp_ref, bp_ref, wr_ref, br_ref,
                    o_ref, pool_ref):
    i = pl.program_id(0)

    @pl.when(i == 0)
    def _():
        pool_ref[...] = jnp.zeros_like(pool_ref)

    seg = seg_ref[0, 0, :]
    gids = lax.broadcasted_iota(jnp.int32, (NGRAPH, ROW_BLK), 0)
    onehot = (gids == seg[None, :]).astype(jnp.float32)
    pool_ref[...] += jnp.dot(onehot, h_ref[...],
                             preferred_element_type=jnp.float32)

    @pl.when(i == GRID - 1)
    def _():
        hp = jnp.maximum(jnp.dot(pool_ref[...], wp_ref[...],
                                 preferred_element_type=jnp.float32)
                         + bp_ref[...], 0.0)
        logits = jnp.dot(hp, wr_ref[...],
                         preferred_element_type=jnp.float32) + br_ref[...]
        m = jnp.max(logits, axis=1, keepdims=True)
        lse = m + jnp.log(jnp.sum(jnp.exp(logits - m), axis=1, keepdims=True))
        o_ref[...] = logits - lse


def _pool_head(h, seg3, wp, bp, wr, br):
    return pl.pallas_call(
        _pool_head_body,
        grid=(GRID,),
        in_specs=[
            pl.BlockSpec((ROW_BLK, D), lambda i: (i, 0)),
            pl.BlockSpec((1, 1, ROW_BLK), lambda i: (i, 0, 0)),
            pl.BlockSpec((D, D), lambda i: (0, 0)),
            pl.BlockSpec((1, D), lambda i: (0, 0)),
            pl.BlockSpec((D, NCLASS), lambda i: (0, 0)),
            pl.BlockSpec((1, NCLASS), lambda i: (0, 0)),
        ],
        out_specs=pl.BlockSpec((NGRAPH, NCLASS), lambda i: (0, 0)),
        out_shape=jax.ShapeDtypeStruct((NGRAPH, NCLASS), jnp.float32),
        scratch_shapes=[pltpu.VMEM((NGRAPH, D), jnp.float32)],
    )(h, seg3, wp, bp.reshape(1, D), wr, br.reshape(1, NCLASS))


def kernel(x, edge_index, batch, W_pre, b_pre, W0a, b0a, W0b, b0b,
           W1a, b1a, W1b, b1b, W2a, b2a, W2b, b2b,
           W_post, b_post, W_read, b_read):
    src = edge_index[0].astype(jnp.int32)
    dst = edge_index[1].astype(jnp.int32)
    pad = E_PAD - E
    src_p = jnp.concatenate(
        [src, jnp.arange(pad, dtype=jnp.int32) % N])
    dst_p = jnp.concatenate(
        [dst, N + (jnp.arange(pad, dtype=jnp.int32) % 64)])
    zeros = jnp.zeros((ACC_ROWS, D), jnp.float32)
    seg3 = batch.astype(jnp.int32).reshape(GRID, 1, ROW_BLK)

    h = _pre_matmul(x, W_pre, b_pre)
    for (wa, ba, wb, bb) in ((W0a, b0a, W0b, b0b),
                             (W1a, b1a, W1b, b1b),
                             (W2a, b2a, W2b, b2b)):
        parts = _sc_agg(h, zeros, src_p, dst_p)
        h = _mlp(parts, wa, ba, wb, bb)
    return _pool_head(h, seg3, W_post, b_post, W_read, b_read)


# speedup vs baseline: 3.6047x; 1.5491x over previous
import functools

import jax
import jax.numpy as jnp
from jax import lax
from jax.experimental import pallas as pl
from jax.experimental.pallas import tpu as pltpu
from jax.experimental.pallas import tpu_sc as plsc

N = 10000
E = 320000
D = 128
NGRAPH = 128
NCLASS = 16

NC = 2
NS = 16
NW = NC * NS
CHUNK = 128
K = 79
KH = K // 2
E_PAD = NW * K * CHUNK
NBUF = 2
ACC_ROWS = N + 64
RPT = 624
LAST_ROWS = N - 15 * RPT
LAST_ZROWS = ACC_ROWS - 15 * RPT

ROW_BLK = 1000
GRID = N // ROW_BLK


def _sc_agg_body(h_hbm, z_hbm, src_hbm, dst_hbm, out_hbm,
                 acc, src_idx, dst_idx, rows, g0, g1):
    gsems = (g0, g1)
    cid = lax.axis_index("c")
    sid = lax.axis_index("s")
    wid = sid * NC + cid

    @pl.when(cid == 0)
    def _():
        @pl.when(sid < 15)
        def _():
            pltpu.sync_copy(h_hbm.at[pl.ds(sid * RPT, RPT)],
                            acc.at[pl.ds(sid * RPT, RPT)])

        @pl.when(sid == 15)
        def _():
            pltpu.sync_copy(h_hbm.at[pl.ds(15 * RPT, LAST_ROWS)],
                            acc.at[pl.ds(15 * RPT, LAST_ROWS)])
            pltpu.sync_copy(z_hbm.at[pl.ds(0, ACC_ROWS - N)],
                            acc.at[pl.ds(N, ACC_ROWS - N)])

    @pl.when(cid == 1)
    def _():
        @pl.when(sid < 15)
        def _():
            pltpu.sync_copy(z_hbm.at[pl.ds(sid * RPT, RPT)],
                            acc.at[pl.ds(sid * RPT, RPT)])

        @pl.when(sid == 15)
        def _():
            pltpu.sync_copy(z_hbm.at[pl.ds(15 * RPT, LAST_ZROWS)],
                            acc.at[pl.ds(15 * RPT, LAST_ZROWS)])

    plsc.subcore_barrier()

    def idx_load(j, b):
        base = (wid * K + j) * CHUNK
        pltpu.sync_copy(src_hbm.at[pl.ds(base, CHUNK)], src_idx.at[b])
        pltpu.sync_copy(dst_hbm.at[pl.ds(base, CHUNK)], dst_idx.at[b])

    def gather(b):
        pltpu.async_copy(h_hbm.at[src_idx.at[b]], rows.at[b], gsems[b])

    def gather_wait(b):
        pltpu.make_async_copy(h_hbm.at[src_idx.at[b]], rows.at[b],
                              gsems[b]).wait()

    idx_load(0, 0)
    gather(0)

    def pair(jj):
        for t in range(2):
            j = jj + t
            b = t
            idx_load(j + 1, 1 - b)
            gather(1 - b)
            gather_wait(b)
            pltpu.sync_copy(rows.at[b], acc.at[dst_idx.at[b]], add=True)

    pl.loop(0, K - 1, step=2)(pair)
    gather_wait(0)
    pltpu.sync_copy(rows.at[0], acc.at[dst_idx.at[0]], add=True)

    plsc.subcore_barrier()

    @pl.when(sid < 15)
    def _():
        pltpu.sync_copy(acc.at[pl.ds(sid * RPT, RPT)],
                        out_hbm.at[cid, pl.ds(sid * RPT, RPT)])

    @pl.when(sid == 15)
    def _():
        pltpu.sync_copy(acc.at[pl.ds(15 * RPT, LAST_ROWS)],
                        out_hbm.at[cid, pl.ds(15 * RPT, LAST_ROWS)])


_sc_agg = pl.kernel(
    _sc_agg_body,
    out_type=jax.ShapeDtypeStruct((NC, N, D), jnp.float32),
    mesh=plsc.VectorSubcoreMesh(core_axis_name="c", subcore_axis_name="s"),
    scratch_types=[
        pltpu.VMEM_SHARED((ACC_ROWS, D), jnp.float32),
        pltpu.VMEM((NBUF, CHUNK), jnp.int32),
        pltpu.VMEM((NBUF, CHUNK), jnp.int32),
        pltpu.VMEM((NBUF, CHUNK, D), jnp.float32),
        pltpu.SemaphoreType.DMA,
        pltpu.SemaphoreType.DMA,
    ],
)


def _pre_body(x_ref, w_ref, b_ref, o_ref):
    o_ref[...] = jnp.dot(x_ref[...], w_ref[...],
                         preferred_element_type=jnp.float32) + b_ref[...]


def _pre_matmul(x, w, b):
    return pl.pallas_call(
        _pre_body,
        grid=(GRID,),
        in_specs=[
            pl.BlockSpec((ROW_BLK, D), lambda i: (i, 0)),
            pl.BlockSpec((D, D), lambda i: (0, 0)),
            pl.BlockSpec((1, D), lambda i: (0, 0)),
        ],
        out_specs=pl.BlockSpec((ROW_BLK, D), lambda i: (i, 0)),
        out_shape=jax.ShapeDtypeStruct((N, D), jnp.float32),
    )(x, w, b.reshape(1, D))


def _mlp_body(p_ref, wa_ref, ba_ref, wb_ref, bb_ref, o_ref):
    t = p_ref[0] + p_ref[1]
    u = jnp.maximum(jnp.dot(t, wa_ref[...],
                            preferred_element_type=jnp.float32) + ba_ref[...],
                    0.0)
    v = jnp.dot(u, wb_ref[...], preferred_element_type=jnp.float32) + bb_ref[...]
    o_ref[...] = jnp.maximum(v, 0.0)


def _mlp(parts, wa, ba, wb, bb):
    return pl.pallas_call(
        _mlp_body,
        grid=(GRID,),
        in_specs=[
            pl.BlockSpec((NC, ROW_BLK, D), lambda i: (0, i, 0)),
            pl.BlockSpec((D, D), lambda i: (0, 0)),
            pl.BlockSpec((1, D), lambda i: (0, 0)),
            pl.BlockSpec((D, D), lambda i: (0, 0)),
            pl.BlockSpec((1, D), lambda i: (0, 0)),
        ],
        out_specs=pl.BlockSpec((ROW_BLK, D), lambda i: (i, 0)),
        out_shape=jax.ShapeDtypeStruct((N, D), jnp.float32),
    )(parts, wa, ba.reshape(1, D), wb, bb.reshape(1, D))


def _pool_head_body(h_ref, seg_ref, wp_ref, bp_ref, wr_ref, br_ref,
                    o_ref, pool_ref):
    i = pl.program_id(0)

    @pl.when(i == 0)
    def _():
        pool_ref[...] = jnp.zeros_like(pool_ref)

    seg = seg_ref[0, 0, :]
    gids = lax.broadcasted_iota(jnp.int32, (NGRAPH, ROW_BLK), 0)
    onehot = (gids == seg[None, :]).astype(jnp.float32)
    pool_ref[...] += jnp.dot(onehot, h_ref[...],
                             preferred_element_type=jnp.float32)

    @pl.when(i == GRID - 1)
    def _():
        hp = jnp.maximum(jnp.dot(pool_ref[...], wp_ref[...],
                                 preferred_element_type=jnp.float32)
                         + bp_ref[...], 0.0)
        logits = jnp.dot(hp, wr_ref[...],
                         preferred_element_type=jnp.float32) + br_ref[...]
        m = jnp.max(logits, axis=1, keepdims=True)
        lse = m + jnp.log(jnp.sum(jnp.exp(logits - m), axis=1, keepdims=True))
        o_ref[...] = logits - lse


def _pool_head(h, seg3, wp, bp, wr, br):
    return pl.pallas_call(
        _pool_head_body,
        grid=(GRID,),
        in_specs=[
            pl.BlockSpec((ROW_BLK, D), lambda i: (i, 0)),
            pl.BlockSpec((1, 1, ROW_BLK), lambda i: (i, 0, 0)),
            pl.BlockSpec((D, D), lambda i: (0, 0)),
            pl.BlockSpec((1, D), lambda i: (0, 0)),
            pl.BlockSpec((D, NCLASS), lambda i: (0, 0)),
            pl.BlockSpec((1, NCLASS), lambda i: (0, 0)),
        ],
        out_specs=pl.BlockSpec((NGRAPH, NCLASS), lambda i: (0, 0)),
        out_shape=jax.ShapeDtypeStruct((NGRAPH, NCLASS), jnp.float32),
        scratch_shapes=[pltpu.VMEM((NGRAPH, D), jnp.float32)],
    )(h, seg3, wp, bp.reshape(1, D), wr, br.reshape(1, NCLASS))


def kernel(x, edge_index, batch, W_pre, b_pre, W0a, b0a, W0b, b0b,
           W1a, b1a, W1b, b1b, W2a, b2a, W2b, b2b,
           W_post, b_post, W_read, b_read):
    src = edge_index[0].astype(jnp.int32)
    dst = edge_index[1].astype(jnp.int32)
    pad = E_PAD - E
    src_p = jnp.concatenate(
        [src, jnp.arange(pad, dtype=jnp.int32) % N])
    dst_p = jnp.concatenate(
        [dst, N + (jnp.arange(pad, dtype=jnp.int32) % 64)])
    zeros = jnp.zeros((ACC_ROWS, D), jnp.float32)
    seg3 = batch.astype(jnp.int32).reshape(GRID, 1, ROW_BLK)

    h = _pre_matmul(x, W_pre, b_pre)
    for (wa, ba, wb, bb) in ((W0a, b0a, W0b, b0b),
                             (W1a, b1a, W1b, b1b),
                             (W2a, b2a, W2b, b2b)):
        parts = _sc_agg(h, zeros, src_p, dst_p)
        h = _mlp(parts, wa, ba, wb, bb)
    return _pool_head(h, seg3, W_post, b_post, W_read, b_read)
